# Initial kernel scaffold; baseline (speedup 1.0000x reference)
import jax
import jax.numpy as jnp
from jax.experimental import pallas as pl


def kernel(x, Wq, Wk, Wv, memory0):
    raise NotImplementedError("write your pallas kernel here")


# speedup vs baseline: 63.6615x; 63.6615x over previous
import jax
import jax.numpy as jnp
from jax.experimental import pallas as pl
from jax.experimental.pallas import tpu as pltpu

_LR = 0.01
_CHUNK = 32


def _la_chunk_kernel(x_ref, wq_ref, wk_ref, wv_ref, m0t_ref, o_ref, z_ref, *, chunk):
    b, _, c, d = x_ref.shape
    nb = b * c

    @pl.when(pl.program_id(0) == 0)
    def _():
        z_ref[...] = m0t_ref[...]

    xb = x_ref[...].reshape(nb, d).astype(jnp.bfloat16)
    q = jnp.dot(xb, wq_ref[...], preferred_element_type=jnp.float32)
    k = jnp.dot(xb, wk_ref[...], preferred_element_type=jnp.float32)
    v = jnp.dot(xb, wv_ref[...], preferred_element_type=jnp.float32)
    qb = q.astype(jnp.bfloat16)
    kb = k.astype(jnp.bfloat16)
    vb = (_LR * v).astype(jnp.bfloat16)

    s = jax.lax.dot_general(qb, kb, (((1,), (1,)), ((), ())),
                            preferred_element_type=jnp.float32)
    ti = jax.lax.broadcasted_iota(jnp.int32, (nb, nb), 0) % c
    tj = jax.lax.broadcasted_iota(jnp.int32, (nb, nb), 1) % c
    sm = jnp.where(ti >= tj, s, 0.0).astype(jnp.bfloat16)

    zb = z_ref[...].astype(jnp.bfloat16)
    out = (jnp.dot(qb, zb, preferred_element_type=jnp.float32)
           + jnp.dot(sm, vb, preferred_element_type=jnp.float32))
    o_ref[...] = out.reshape(b, 1, c, d)

    z_ref[...] += jax.lax.dot_general(kb, vb, (((0,), (0,)), ((), ())),
                                      preferred_element_type=jnp.float32)


def kernel(x, Wq, Wk, Wv, memory0, *, chunk=_CHUNK, interpret=False):
    B, T, D = x.shape
    n_chunks = T // chunk
    x4 = x.reshape(B, n_chunks, chunk, D)
    wqt = Wq.T.astype(jnp.bfloat16)
    wkt = Wk.T.astype(jnp.bfloat16)
    wvt = Wv.T.astype(jnp.bfloat16)
    m0t = memory0.T

    import functools
    body = functools.partial(_la_chunk_kernel, chunk=chunk)

    out = pl.pallas_call(
        body,
        out_shape=jax.ShapeDtypeStruct((B, n_chunks, chunk, D), jnp.float32),
        grid=(n_chunks,),
        in_specs=[
            pl.BlockSpec((B, 1, chunk, D), lambda c: (0, c, 0, 0)),
            pl.BlockSpec((D, D), lambda c: (0, 0)),
            pl.BlockSpec((D, D), lambda c: (0, 0)),
            pl.BlockSpec((D, D), lambda c: (0, 0)),
            pl.BlockSpec((D, D), lambda c: (0, 0)),
        ],
        out_specs=pl.BlockSpec((B, 1, chunk, D), lambda c: (0, c, 0, 0)),
        scratch_shapes=[pltpu.VMEM((D, D), jnp.float32)],
        compiler_params=pltpu.CompilerParams(
            dimension_semantics=("arbitrary",),
        ),
        name="linear_attention_chunked",
        interpret=interpret,
    )(x4, wqt, wkt, wvt, m0t)
    return out.reshape(B, T, D)
